# baseline (speedup 1.0000x reference)
import jax
import jax.numpy as jnp
from jax.experimental import pallas as pl

N = 50000
E = 800000
D = 100


def _conv(x, src, dst, Wq, bq, Wk, bk, Wv, bv, Ws, bs):
    q = x @ Wq + bq
    k = x @ Wk + bk
    v = x @ Wv + bv
    qi = q[dst]
    kj = k[src]
    vj = v[src]
    score = jnp.sum(qi * kj, axis=-1) / jnp.sqrt(jnp.float32(D))
    m = jax.ops.segment_max(score, dst, num_segments=N)
    e = jnp.exp(score - m[dst])
    s = jax.ops.segment_sum(e, dst, num_segments=N)
    alpha = e / s[dst]
    agg = jax.ops.segment_sum(alpha[:, None] * vj, dst, num_segments=N)
    out = agg + x @ Ws + bs
    return out, alpha


def _bn(x, gamma, beta):
    mean = jnp.mean(x, axis=0)
    var = jnp.var(x, axis=0)
    return gamma * (x - mean) * jax.lax.rsqrt(var + 1e-5) + beta


def _block(x0, src, dst, p):
    x, a1 = _conv(x0, src, dst,
                  p["conv1_q_W"], p["conv1_q_b"],
                  p["conv1_k_W"], p["conv1_k_b"],
                  p["conv1_v_W"], p["conv1_v_b"],
                  p["conv1_skip_W"], p["conv1_skip_b"])
    x = jax.nn.relu(_bn(x, p["bn1_gamma"], p["bn1_beta"]))
    x, a2 = _conv(x, src, dst,
                  p["conv2_q_W"], p["conv2_q_b"],
                  p["conv2_k_W"], p["conv2_k_b"],
                  p["conv2_v_W"], p["conv2_v_b"],
                  p["conv2_skip_W"], p["conv2_skip_b"])
    x = _bn(x, p["bn2_gamma"], p["bn2_beta"])
    x = jax.nn.relu(x + x0)
    return x, a1 + a2


def kernel(x, edges, params):
    src = edges[0]
    dst = edges[1]
    x, a0 = _block(x, src, dst, params[0])
    x, a1 = _block(x, src, dst, params[1])
    x, a2 = _block(x, src, dst, params[2])
    att = (a0 + a1 + a2)[:, None]
    return (x, (edges, att))
